# baseline (speedup 1.0000x reference)
import functools

import jax
import jax.numpy as jnp
from jax import lax
from jax.experimental import pallas as pl
from jax.experimental.pallas import tpu as pltpu
from jax.experimental.pallas import tpu_sc as plsc

B_Q = 1024
D = 32
N_U = 100000
BLK = 2000
NB = N_U // BLK
KK = 6
TOPK = 5
NC = 8
CAND = NB * NC
SENT = -2.0
BIG = 1 << 30


def _scan_body(q_ref, u_ref, un_ref, cv3_ref, ci3_ref, qn_ref, thr_ref):
    cv_ref = cv3_ref.at[0]
    ci_ref = ci3_ref.at[0]
    j = pl.program_id(0)

    @pl.when(j == 0)
    def _init():
        q = q_ref[:, :]
        n = jnp.sqrt(jnp.sum(q * q, axis=1, keepdims=True))
        qn_ref[:, :] = q / jnp.maximum(n, 1e-12)
        thr_ref[:, :] = jnp.full((B_Q, 1), SENT, jnp.float32)

    u = u_ref[:, :]
    n = jnp.sqrt(jnp.sum(u * u, axis=1, keepdims=True))
    un = u / jnp.maximum(n, 1e-12)
    un_ref[:, :] = un

    qn = qn_ref[:, :]
    s = lax.dot_general(qn, un, (((1,), (1,)), ((), ())),
                        preferred_element_type=jnp.float32)

    thr = thr_ref[:, :]
    base = j * BLK
    cnt = jnp.sum((s >= thr).astype(jnp.int32), axis=1, keepdims=True)
    k_needed = jnp.minimum(jnp.max(cnt), KK)

    cv_ref[:, :] = jnp.full((B_Q, NC), SENT, jnp.float32)
    ci_ref[:, :] = jnp.zeros((B_Q, NC), jnp.int32)

    def _case(k):
        def body():
            sl = s
            iota = lax.broadcasted_iota(jnp.int32, (B_Q, BLK), 1)
            for t in range(k):
                m = jnp.max(sl, axis=1, keepdims=True)
                a = jnp.min(jnp.where(sl == m, iota, BIG), axis=1,
                            keepdims=True)
                cv_ref[:, t:t + 1] = m
                ci_ref[:, t:t + 1] = a + base
                if t < k - 1:
                    sl = jnp.where(iota == a, SENT, sl)
        return body

    lax.switch(k_needed, [_case(k) for k in range(KK + 1)])
    thr_ref[:, :] = jnp.maximum(thr, cv_ref[:, KK - 1:KK])


def _merge_body(cv_ref, ci_ref, vals_ref, idx_ref):
    cv = cv_ref[:, :]
    ci = ci_ref[:, :]
    piota = lax.broadcasted_iota(jnp.int32, (B_Q, CAND), 1)
    nv, ni = [], []
    for _ in range(KK):
        m = jnp.max(cv, axis=1, keepdims=True)
        p = jnp.min(jnp.where(cv == m, piota, BIG), axis=1, keepdims=True)
        nv.append(m)
        ni.append(jnp.max(jnp.where(piota == p, ci, -1), axis=1,
                          keepdims=True))
        cv = jnp.where(piota == p, SENT, cv)
    v = jnp.concatenate(nv, axis=1)
    ii = jnp.concatenate(ni, axis=1)

    mask = v < 0.9999
    mi = mask.astype(jnp.int32)
    run = jnp.zeros((B_Q, 1), jnp.int32)
    excl_l = []
    for p_ in range(KK):
        excl_l.append(run)
        run = run + mi[:, p_:p_ + 1]
    excl = jnp.concatenate(excl_l, axis=1)
    count = run
    iota6 = lax.broadcasted_iota(jnp.int32, (B_Q, KK), 1)
    dest = jnp.where(mask, excl, count + (iota6 - excl))
    cm1 = jnp.maximum(count - 1, 0)
    sv, si = [], []
    for p_ in range(TOPK):
        gp = jnp.minimum(jnp.int32(p_), cm1)
        hit = dest == gp
        sv.append(jnp.sum(jnp.where(hit, v, 0.0), axis=1, keepdims=True))
        si.append(jnp.sum(jnp.where(hit, ii, 0), axis=1, keepdims=True))
    sel_v = jnp.concatenate(sv, axis=1)
    sel_i = jnp.concatenate(si, axis=1)
    empty = count == 0
    vals_ref[:, :] = jnp.where(empty, v[:, 0:TOPK], sel_v)
    idx_ref[:, :] = jnp.where(empty, ii[:, 0:TOPK], sel_i)


def _scan_pallas(q, u, interpret=False):
    return pl.pallas_call(
        _scan_body,
        grid=(NB,),
        in_specs=[
            pl.BlockSpec((B_Q, D), lambda j: (0, 0)),
            pl.BlockSpec((BLK, D), lambda j: (j, 0)),
        ],
        out_specs=[
            pl.BlockSpec((BLK, D), lambda j: (j, 0)),
            pl.BlockSpec((1, B_Q, NC), lambda j: (j, 0, 0)),
            pl.BlockSpec((1, B_Q, NC), lambda j: (j, 0, 0)),
        ],
        out_shape=[
            jax.ShapeDtypeStruct((N_U, D), jnp.float32),
            jax.ShapeDtypeStruct((NB, B_Q, NC), jnp.float32),
            jax.ShapeDtypeStruct((NB, B_Q, NC), jnp.int32),
        ],
        scratch_shapes=[
            pltpu.VMEM((B_Q, D), jnp.float32),
            pltpu.VMEM((B_Q, 1), jnp.float32),
        ],
        interpret=interpret,
    )(q, u)


def _merge_pallas(cv, ci, interpret=False):
    return pl.pallas_call(
        _merge_body,
        out_shape=[
            jax.ShapeDtypeStruct((B_Q, TOPK), jnp.float32),
            jax.ShapeDtypeStruct((B_Q, TOPK), jnp.int32),
        ],
        interpret=interpret,
    )(cv, ci)


@functools.cache
def _make_gather():
    info = plsc.get_sparse_core_info()
    nw = info.num_cores * info.num_subcores
    total = B_Q * TOPK
    b_per_w = total // nw
    mesh = plsc.VectorSubcoreMesh(core_axis_name="c", subcore_axis_name="s")

    @functools.partial(
        pl.kernel,
        out_type=jax.ShapeDtypeStruct((total, D), jnp.float32),
        mesh=mesh,
        compiler_params=pltpu.CompilerParams(use_tc_tiling_on_sc=False),
        scratch_types=[
            pltpu.VMEM((b_per_w,), jnp.int32),
            pltpu.VMEM((b_per_w, D), jnp.float32),
            pltpu.SemaphoreType.DMA,
        ],
    )
    def gather_k(table_hbm, idx_hbm, out_hbm, idx_v, rows_v, sem):
        wid = lax.axis_index("s") * info.num_cores + lax.axis_index("c")
        base = wid * b_per_w
        pltpu.sync_copy(idx_hbm.at[pl.ds(base, b_per_w)], idx_v)
        pltpu.async_copy(table_hbm.at[idx_v], rows_v, sem).wait()
        pltpu.sync_copy(rows_v, out_hbm.at[pl.ds(base, b_per_w)])

    return gather_k


def kernel(query_embeddings, user_embeddings):
    un, cv3, ci3 = _scan_pallas(query_embeddings, user_embeddings)
    cv = cv3.transpose(1, 0, 2).reshape(B_Q, CAND)
    ci = ci3.transpose(1, 0, 2).reshape(B_Q, CAND)
    vals, idx = _merge_pallas(cv, ci)
    rows = _make_gather()(un, idx.reshape(-1))
    return rows.reshape(B_Q, TOPK, D), vals


# speedup vs baseline: 2.9525x; 2.9525x over previous
import functools

import jax
import jax.numpy as jnp
from jax import lax
from jax.experimental import pallas as pl
from jax.experimental.pallas import tpu as pltpu
from jax.experimental.pallas import tpu_sc as plsc

B_Q = 1024
D = 32
N_U = 100000
BLK = 2000
NB = N_U // BLK
KK = 6
TOPK = 5
NC = 8
CAND = NB * NC
SENT = -2.0
BIG = 1 << 30


def _scan_body(q_ref, u_ref, un_ref, cv3_ref, ci3_ref, qn_ref):
    cv_ref = cv3_ref.at[0]
    ci_ref = ci3_ref.at[0]
    j = pl.program_id(0)

    @pl.when(j == 0)
    def _init():
        q = q_ref[:, :]
        n = jnp.sqrt(jnp.sum(q * q, axis=1, keepdims=True))
        qn_ref[:, :] = q / jnp.maximum(n, 1e-12)

    u = u_ref[:, :]
    n = jnp.sqrt(jnp.sum(u * u, axis=1, keepdims=True))
    un = u / jnp.maximum(n, 1e-12)
    un_ref[:, :] = un

    qn = qn_ref[:, :]
    s = lax.dot_general(qn, un, (((1,), (1,)), ((), ())),
                        preferred_element_type=jnp.float32)

    base = j * BLK
    iota = lax.broadcasted_iota(jnp.int32, (B_Q, BLK), 1)
    sl = s
    bv, bi = [], []
    for t in range(KK):
        m = jnp.max(sl, axis=1, keepdims=True)
        a = jnp.min(jnp.where(sl == m, iota, BIG), axis=1, keepdims=True)
        bv.append(m)
        bi.append(a + base)
        if t < KK - 1:
            sl = jnp.where(iota == a, SENT, sl)
    cv_ref[:, :] = jnp.concatenate(bv + [jnp.full((B_Q, NC - KK), SENT,
                                                  jnp.float32)], axis=1)
    ci_ref[:, :] = jnp.concatenate(bi + [jnp.zeros((B_Q, NC - KK),
                                                   jnp.int32)], axis=1)


def _merge_body(cv_ref, ci_ref, vals_ref, idx_ref):
    cv = cv_ref[:, :]
    ci = ci_ref[:, :]
    piota = lax.broadcasted_iota(jnp.int32, (B_Q, CAND), 1)
    nv, ni = [], []
    for _ in range(KK):
        m = jnp.max(cv, axis=1, keepdims=True)
        p = jnp.min(jnp.where(cv == m, piota, BIG), axis=1, keepdims=True)
        nv.append(m)
        ni.append(jnp.max(jnp.where(piota == p, ci, -1), axis=1,
                          keepdims=True))
        cv = jnp.where(piota == p, SENT, cv)
    v = jnp.concatenate(nv, axis=1)
    ii = jnp.concatenate(ni, axis=1)

    mask = v < 0.9999
    mi = mask.astype(jnp.int32)
    run = jnp.zeros((B_Q, 1), jnp.int32)
    excl_l = []
    for p_ in range(KK):
        excl_l.append(run)
        run = run + mi[:, p_:p_ + 1]
    excl = jnp.concatenate(excl_l, axis=1)
    count = run
    iota6 = lax.broadcasted_iota(jnp.int32, (B_Q, KK), 1)
    dest = jnp.where(mask, excl, count + (iota6 - excl))
    cm1 = jnp.maximum(count - 1, 0)
    sv, si = [], []
    for p_ in range(TOPK):
        gp = jnp.minimum(jnp.int32(p_), cm1)
        hit = dest == gp
        sv.append(jnp.sum(jnp.where(hit, v, 0.0), axis=1, keepdims=True))
        si.append(jnp.sum(jnp.where(hit, ii, 0), axis=1, keepdims=True))
    sel_v = jnp.concatenate(sv, axis=1)
    sel_i = jnp.concatenate(si, axis=1)
    empty = count == 0
    vals_ref[:, :] = jnp.where(empty, v[:, 0:TOPK], sel_v)
    idx_ref[:, :] = jnp.where(empty, ii[:, 0:TOPK], sel_i)


def _scan_pallas(q, u, interpret=False):
    return pl.pallas_call(
        _scan_body,
        grid=(NB,),
        in_specs=[
            pl.BlockSpec((B_Q, D), lambda j: (0, 0)),
            pl.BlockSpec((BLK, D), lambda j: (j, 0)),
        ],
        out_specs=[
            pl.BlockSpec((BLK, D), lambda j: (j, 0)),
            pl.BlockSpec((1, B_Q, NC), lambda j: (j, 0, 0)),
            pl.BlockSpec((1, B_Q, NC), lambda j: (j, 0, 0)),
        ],
        out_shape=[
            jax.ShapeDtypeStruct((N_U, D), jnp.float32),
            jax.ShapeDtypeStruct((NB, B_Q, NC), jnp.float32),
            jax.ShapeDtypeStruct((NB, B_Q, NC), jnp.int32),
        ],
        scratch_shapes=[
            pltpu.VMEM((B_Q, D), jnp.float32),
        ],
        interpret=interpret,
    )(q, u)


def _merge_pallas(cv, ci, interpret=False):
    return pl.pallas_call(
        _merge_body,
        out_shape=[
            jax.ShapeDtypeStruct((B_Q, TOPK), jnp.float32),
            jax.ShapeDtypeStruct((B_Q, TOPK), jnp.int32),
        ],
        interpret=interpret,
    )(cv, ci)


@functools.cache
def _make_gather():
    info = plsc.get_sparse_core_info()
    nw = info.num_cores * info.num_subcores
    total = B_Q * TOPK
    b_per_w = total // nw
    mesh = plsc.VectorSubcoreMesh(core_axis_name="c", subcore_axis_name="s")

    @functools.partial(
        pl.kernel,
        out_type=jax.ShapeDtypeStruct((total, D), jnp.float32),
        mesh=mesh,
        compiler_params=pltpu.CompilerParams(use_tc_tiling_on_sc=False),
        scratch_types=[
            pltpu.VMEM((b_per_w,), jnp.int32),
            pltpu.VMEM((b_per_w, D), jnp.float32),
            pltpu.SemaphoreType.DMA,
        ],
    )
    def gather_k(table_hbm, idx_hbm, out_hbm, idx_v, rows_v, sem):
        wid = lax.axis_index("s") * info.num_cores + lax.axis_index("c")
        base = wid * b_per_w
        pltpu.sync_copy(idx_hbm.at[pl.ds(base, b_per_w)], idx_v)
        pltpu.async_copy(table_hbm.at[idx_v], rows_v, sem).wait()
        pltpu.sync_copy(rows_v, out_hbm.at[pl.ds(base, b_per_w)])

    return gather_k


def kernel(query_embeddings, user_embeddings):
    un, cv3, ci3 = _scan_pallas(query_embeddings, user_embeddings)
    cv = cv3.transpose(1, 0, 2).reshape(B_Q, CAND)
    ci = ci3.transpose(1, 0, 2).reshape(B_Q, CAND)
    vals, idx = _merge_pallas(cv, ci)
    rows = _make_gather()(un, idx.reshape(-1))
    return rows.reshape(B_Q, TOPK, D), vals
